# baseline (speedup 1.0000x reference)
import functools

import jax
import jax.numpy as jnp
from jax import lax
from jax.experimental import pallas as pl
from jax.experimental.pallas import tpu as pltpu
from jax.experimental.pallas import tpu_sc as plsc

_NC = 2
_NS = 16
_NW = _NC * _NS

_B = 4096
_L = 50
_H = 32
_V = 1000000
_TOT = _B * _L
_PER_W = _TOT // _NW
_CH = 128
_NCH = _PER_W // _CH

_GW = 128
_NFULL = _V // _GW
_REM = _V - _NFULL * _GW
_K1_PER_W = _NFULL // _NW
_K1_EXTRA = _NFULL - _K1_PER_W * _NW
_SROWS = _V // 4


def _transpose_vmem(vin, vout, i16):
    row_lo = i16
    row_hi = i16 + 16

    def group(g, carry):
        colv = i16 * 0 + 16 * g
        for jj in range(16):
            lo = plsc.load_gather(vin, [row_lo, colv + jj])
            hi = plsc.load_gather(vin, [row_hi, colv + jj])
            vout[4 * g + jj // 4, pl.ds((jj % 4) * 32, 16)] = lo
            vout[4 * g + jj // 4, pl.ds((jj % 4) * 32 + 16, 16)] = hi
        return carry

    lax.fori_loop(0, 8, group, 0)


def _k1_body(tab_t, tail, scratch_hbm, vin, vout):
    wid = lax.axis_index("s") * _NC + lax.axis_index("c")
    i16 = lax.iota(jnp.int32, 16)

    def slab(c):
        col0 = pl.multiple_of(_GW * c, _GW)
        row0 = pl.multiple_of(32 * c, 32)
        pltpu.sync_copy(tab_t.at[:, pl.ds(col0, _GW)], vin)
        _transpose_vmem(vin, vout, i16)
        pltpu.sync_copy(vout, scratch_hbm.at[pl.ds(row0, 32)])

    def step(k, carry):
        slab(wid + _NW * k)
        return carry

    lax.fori_loop(0, _K1_PER_W, step, 0)

    @pl.when(wid < _K1_EXTRA)
    def _():
        slab(_K1_PER_W * _NW + wid)

    @pl.when(wid == _K1_EXTRA)
    def _():
        pltpu.sync_copy(tail, vin)
        _transpose_vmem(vin, vout, i16)
        pltpu.sync_copy(vout.at[pl.ds(0, 16)],
                        scratch_hbm.at[pl.ds(_SROWS - 16, 16)])


def _k2_body(cond_t, scratch_hbm, pat_hbm, out_hbm,
             vcond, vpat, vg, vo, vgath, vrows, sem):
    wid = lax.axis_index("s") * _NC + lax.axis_index("c")
    i16 = lax.iota(jnp.int32, 16)
    pltpu.sync_copy(cond_t.at[:, pl.ds(_CH * wid, _CH)], vcond)
    pltpu.sync_copy(pat_hbm, vpat)
    def build(r, carry):
        for k in range(8):
            pvec = vpat[r, pl.ds(16 * k, 16)]
            idx = plsc.load_gather(vcond, [pvec >> 7, pvec & 127])
            vg[r, pl.ds(16 * k, 16)] = idx >> 2
            vo[r, pl.ds(16 * k, 16)] = (idx & 3) << 5
        return carry

    lax.fori_loop(0, _PER_W // _CH, build, 0)
    base = _PER_W // 4 * wid

    def chunk(j, carry):
        pltpu.async_copy(scratch_hbm.at[vg.at[j]], vgath, sem).wait()
        ovecs = [vo[j, pl.ds(q0, 16)] for q0 in range(0, _CH, 16)]

        def hstep(h, carry2):
            for qi in range(_CH // 16):
                q0 = 16 * qi
                x = plsc.load_gather(vgath, [q0 + i16, ovecs[qi] + h])
                plsc.store_scatter(
                    vrows, [(q0 + i16) >> 2, ((q0 + i16) & 3) * 32 + h], x)
            return carry2

        lax.fori_loop(0, _H, hstep, 0)
        pltpu.sync_copy(vrows, out_hbm.at[pl.ds(base + 32 * j, 32)])
        return carry

    lax.fori_loop(0, _NCH, chunk, 0)


@jax.jit
def kernel(conditions, table):
    tab_t = table.T
    cond_t = conditions.T
    tail = jnp.pad(lax.slice(tab_t, (0, _V - _REM), (_H, _V)),
                   ((0, 0), (0, _GW - _REM)))
    p = jnp.arange(_PER_W, dtype=jnp.int32)
    pattern = ((p % _L) * _CH + p // _L).reshape(_L, _CH)

    mesh = plsc.VectorSubcoreMesh(
        core_axis_name="c", subcore_axis_name="s",
        num_cores=_NC, num_subcores=_NS)
    params = pltpu.CompilerParams(use_tc_tiling_on_sc=True,
                                  needs_layout_passes=False)

    scratch = pl.kernel(
        _k1_body,
        out_type=jax.ShapeDtypeStruct((_SROWS, 128), jnp.float32),
        mesh=mesh,
        scratch_types=[
            pltpu.VMEM((_H, 128), jnp.float32),
            pltpu.VMEM((_H, 128), jnp.float32),
        ],
        compiler_params=params,
    )(tab_t, tail)

    out4 = pl.kernel(
        _k2_body,
        out_type=jax.ShapeDtypeStruct((_TOT // 4, 128), jnp.float32),
        mesh=mesh,
        scratch_types=[
            pltpu.VMEM((_L, _CH), jnp.int32),
            pltpu.VMEM((_L, _CH), jnp.int32),
            pltpu.VMEM((_NCH, _CH), jnp.int32),
            pltpu.VMEM((_NCH, _CH), jnp.int32),
            pltpu.VMEM((_CH, 128), jnp.float32),
            pltpu.VMEM((_CH // 4, 128), jnp.float32),
            pltpu.SemaphoreType.DMA,
        ],
        compiler_params=params,
    )(cond_t, scratch, pattern)

    return out4.reshape(_B, _L * _H)


# speedup vs baseline: 1.3393x; 1.3393x over previous
import functools

import jax
import jax.numpy as jnp
from jax import lax
from jax.experimental import pallas as pl
from jax.experimental.pallas import tpu as pltpu
from jax.experimental.pallas import tpu_sc as plsc

_NC = 2
_NS = 16
_NW = _NC * _NS

_B = 4096
_L = 50
_H = 32
_V = 1000000
_TOT = _B * _L
_PER_W = _TOT // _NW
_CH = 128
_NCH = _PER_W // _CH

_GW = 128
_NFULL = _V // _GW
_REM = _V - _NFULL * _GW
_K1_PER_W = _NFULL // _NW
_K1_EXTRA = _NFULL - _K1_PER_W * _NW
_SROWS = _V // 4


def _transpose_vmem(vin, vout, i16):
    row_lo = i16
    row_hi = i16 + 16

    def group(g, carry):
        colv = i16 * 0 + 16 * g
        for jj in range(16):
            lo = plsc.load_gather(vin, [row_lo, colv + jj])
            hi = plsc.load_gather(vin, [row_hi, colv + jj])
            vout[4 * g + jj // 4, pl.ds((jj % 4) * 32, 16)] = lo
            vout[4 * g + jj // 4, pl.ds((jj % 4) * 32 + 16, 16)] = hi
        return carry

    lax.fori_loop(0, 8, group, 0)


def _k1_body(tab_t, tail, scratch_hbm,
             vin0, vin1, vin2, vin3, vout0, vout1,
             gs0, gs1, gs2, gs3, ws0, ws1):
    wid = lax.axis_index("s") * _NC + lax.axis_index("c")
    i16 = lax.iota(jnp.int32, 16)
    vins = [vin0, vin1, vin2, vin3]
    vouts = [vout0, vout1]
    gsems = [gs0, gs1, gs2, gs3]
    wsems = [ws0, ws1]

    def col(s):
        return pl.multiple_of(_GW * (wid + _NW * s), _GW)

    def row(s):
        return pl.multiple_of(32 * (wid + _NW * s), 32)

    def fire_in(a, s):
        pltpu.async_copy(tab_t.at[:, pl.ds(col(s), _GW)], vins[a], gsems[a])

    def wait_in(a):
        pltpu.make_async_copy(tab_t.at[:, pl.ds(0, _GW)], vins[a],
                              gsems[a]).wait()

    def fire_write(b, s):
        pltpu.async_copy(vouts[b], scratch_hbm.at[pl.ds(row(s), 32)],
                         wsems[b])

    def wait_write(b):
        pltpu.make_async_copy(vouts[b], scratch_hbm.at[pl.ds(0, 32)],
                              wsems[b]).wait()

    for a in range(4):
        fire_in(a, a)

    def group(t, carry):
        for a in range(4):
            s = 4 * t + a
            b = a % 2
            wait_in(a)

            @pl.when(s >= 2)
            def _():
                wait_write(b)
            _transpose_vmem(vins[a], vouts[b], i16)
            fire_write(b, s)

            @pl.when(s + 4 < _K1_PER_W)
            def _():
                fire_in(a, s + 4)
        return carry

    lax.fori_loop(0, _K1_PER_W // 4, group, 0)
    wait_write(0)
    wait_write(1)

    @pl.when(wid < _K1_EXTRA)
    def _():
        c = _K1_PER_W * _NW + wid
        pltpu.sync_copy(tab_t.at[:, pl.ds(pl.multiple_of(_GW * c, _GW),
                                          _GW)], vin0)
        _transpose_vmem(vin0, vout0, i16)
        pltpu.sync_copy(vout0, scratch_hbm.at[pl.ds(32 * c, 32)])

    @pl.when(wid == _K1_EXTRA)
    def _():
        pltpu.sync_copy(tail, vin0)
        _transpose_vmem(vin0, vout0, i16)
        pltpu.sync_copy(vout0.at[pl.ds(0, 16)],
                        scratch_hbm.at[pl.ds(_SROWS - 16, 16)])


def _k2_body(cond_t, scratch_hbm, pat_hbm, out_hbm,
             vcond, vpat, vg, vo, vgath0, vgath1, vrows0, vrows1,
             gs0, gs1, ws0, ws1):
    wid = lax.axis_index("s") * _NC + lax.axis_index("c")
    i16 = lax.iota(jnp.int32, 16)
    vgaths = [vgath0, vgath1]
    vrows = [vrows0, vrows1]
    gsems = [gs0, gs1]
    wsems = [ws0, ws1]
    pltpu.sync_copy(cond_t.at[:, pl.ds(_CH * wid, _CH)], vcond)
    pltpu.sync_copy(pat_hbm, vpat)

    def build(r, carry):
        for k in range(8):
            pvec = vpat[r, pl.ds(16 * k, 16)]
            idx = plsc.load_gather(vcond, [pvec >> 7, pvec & 127])
            vg[r, pl.ds(16 * k, 16)] = idx >> 2
            vo[r, pl.ds(16 * k, 16)] = (idx & 3) << 5
        return carry

    lax.fori_loop(0, _PER_W // _CH, build, 0)
    base = _PER_W // 4 * wid

    def fire_gather(b, j):
        pltpu.async_copy(scratch_hbm.at[vg.at[j]], vgaths[b], gsems[b])

    def wait_gather(b):
        pltpu.make_async_copy(scratch_hbm.at[pl.ds(0, _CH)], vgaths[b],
                              gsems[b]).wait()

    def fire_write(b, j):
        pltpu.async_copy(vrows[b], out_hbm.at[pl.ds(base + 32 * j, 32)],
                         wsems[b])

    def wait_write(b):
        pltpu.make_async_copy(vrows[b], out_hbm.at[pl.ds(0, 32)],
                              wsems[b]).wait()

    fire_gather(0, 0)
    fire_gather(1, 1)

    def pair(t, carry):
        for b in range(2):
            j = 2 * t + b
            wait_gather(b)

            @pl.when(j >= 2)
            def _():
                wait_write(b)
            ovecs = [vo[j, pl.ds(q0, 16)] for q0 in range(0, _CH, 16)]

            def hstep(h, carry2):
                for qi in range(_CH // 16):
                    q0 = 16 * qi
                    x = plsc.load_gather(vgaths[b],
                                         [q0 + i16, ovecs[qi] + h])
                    plsc.store_scatter(
                        vrows[b],
                        [(q0 + i16) >> 2, ((q0 + i16) & 3) * 32 + h], x)
                return carry2

            lax.fori_loop(0, _H, hstep, 0)
            fire_write(b, j)

            @pl.when(j + 2 < _NCH)
            def _():
                fire_gather(b, j + 2)
        return carry

    lax.fori_loop(0, _NCH // 2, pair, 0)
    wait_write(0)
    wait_write(1)


@jax.jit
def kernel(conditions, table):
    tab_t = table.T
    cond_t = conditions.T
    tail = jnp.pad(lax.slice(tab_t, (0, _V - _REM), (_H, _V)),
                   ((0, 0), (0, _GW - _REM)))
    p = jnp.arange(_PER_W, dtype=jnp.int32)
    pattern = ((p % _L) * _CH + p // _L).reshape(_L, _CH)

    mesh = plsc.VectorSubcoreMesh(
        core_axis_name="c", subcore_axis_name="s",
        num_cores=_NC, num_subcores=_NS)
    params = pltpu.CompilerParams(use_tc_tiling_on_sc=True,
                                  needs_layout_passes=False)

    scratch = pl.kernel(
        _k1_body,
        out_type=jax.ShapeDtypeStruct((_SROWS, 128), jnp.float32),
        mesh=mesh,
        scratch_types=(
            [pltpu.VMEM((_H, 128), jnp.float32) for _ in range(6)]
            + [pltpu.SemaphoreType.DMA for _ in range(6)]
        ),
        compiler_params=params,
    )(tab_t, tail)

    out4 = pl.kernel(
        _k2_body,
        out_type=jax.ShapeDtypeStruct((_TOT // 4, 128), jnp.float32),
        mesh=mesh,
        scratch_types=[
            pltpu.VMEM((_L, _CH), jnp.int32),
            pltpu.VMEM((_L, _CH), jnp.int32),
            pltpu.VMEM((_NCH, _CH), jnp.int32),
            pltpu.VMEM((_NCH, _CH), jnp.int32),
            pltpu.VMEM((_CH, 128), jnp.float32),
            pltpu.VMEM((_CH, 128), jnp.float32),
            pltpu.VMEM((_CH // 4, 128), jnp.float32),
            pltpu.VMEM((_CH // 4, 128), jnp.float32),
            pltpu.SemaphoreType.DMA,
            pltpu.SemaphoreType.DMA,
            pltpu.SemaphoreType.DMA,
            pltpu.SemaphoreType.DMA,
        ],
        compiler_params=params,
    )(cond_t, scratch, pattern)

    return out4.reshape(_B, _L * _H)


# speedup vs baseline: 1.6466x; 1.2294x over previous
import functools

import jax
import jax.numpy as jnp
from jax import lax
from jax.experimental import pallas as pl
from jax.experimental.pallas import tpu as pltpu
from jax.experimental.pallas import tpu_sc as plsc

_NC = 2
_NS = 16
_NW = _NC * _NS

_B = 4096
_L = 50
_H = 32
_V = 1000000
_TOT = _B * _L
_PER_W = _TOT // _NW
_CH = 128
_NCH = _PER_W // _CH

_GW = 128
_NFULL = _V // _GW
_REM = _V - _NFULL * _GW
_K1_PER_W = _NFULL // _NW
_K1_EXTRA = _NFULL - _K1_PER_W * _NW
_SROWS = _V // 4


def _k2_body(cond_t, scratch_hbm, pat_hbm, out_hbm,
             vcond, vpat, vg, vo, vgath0, vgath1, vrows0, vrows1,
             gs0, gs1, ws0, ws1):
    wid = lax.axis_index("s") * _NC + lax.axis_index("c")
    i16 = lax.iota(jnp.int32, 16)
    vgaths = [vgath0, vgath1]
    vrows = [vrows0, vrows1]
    gsems = [gs0, gs1]
    wsems = [ws0, ws1]
    pltpu.sync_copy(cond_t.at[:, pl.ds(_CH * wid, _CH)], vcond)
    pltpu.sync_copy(pat_hbm, vpat)

    def build(r, carry):
        for k in range(8):
            pvec = vpat[r, pl.ds(16 * k, 16)]
            idx = plsc.load_gather(vcond, [pvec >> 7, pvec & 127])
            vg[r, pl.ds(16 * k, 16)] = idx >> 2
            vo[r, pl.ds(16 * k, 16)] = (idx & 3) << 5
        return carry

    lax.fori_loop(0, _PER_W // _CH, build, 0)
    base = _PER_W // 4 * wid

    def fire_gather(b, j):
        pltpu.async_copy(scratch_hbm.at[vg.at[j]], vgaths[b], gsems[b])

    def wait_gather(b):
        pltpu.make_async_copy(scratch_hbm.at[pl.ds(0, _CH)], vgaths[b],
                              gsems[b]).wait()

    def fire_write(b, j):
        pltpu.async_copy(vrows[b], out_hbm.at[pl.ds(base + 32 * j, 32)],
                         wsems[b])

    def wait_write(b):
        pltpu.make_async_copy(vrows[b], out_hbm.at[pl.ds(0, 32)],
                              wsems[b]).wait()

    fire_gather(0, 0)
    fire_gather(1, 1)

    def pair(t, carry):
        for b in range(2):
            j = 2 * t + b
            wait_gather(b)

            @pl.when(j >= 2)
            def _():
                wait_write(b)
            ovecs = [vo[j, pl.ds(q0, 16)] for q0 in range(0, _CH, 16)]
            qrows = [(q0 + i16) >> 2 for q0 in range(0, _CH, 16)]
            qcols = [((q0 + i16) & 3) * 32 for q0 in range(0, _CH, 16)]
            grows = [q0 + i16 for q0 in range(0, _CH, 16)]

            def hstep(h, carry2):
                for qi in range(_CH // 16):
                    x = plsc.load_gather(vgaths[b],
                                         [grows[qi], ovecs[qi] + h])
                    plsc.store_scatter(vrows[b], [qrows[qi], qcols[qi] + h],
                                       x)
                return carry2

            lax.fori_loop(0, _H, hstep, 0)
            fire_write(b, j)

            @pl.when(j + 2 < _NCH)
            def _():
                fire_gather(b, j + 2)
        return carry

    lax.fori_loop(0, _NCH // 2, pair, 0)
    wait_write(0)
    wait_write(1)


@jax.jit
def kernel(conditions, table):
    cond_t = conditions.T
    scratch = jnp.reshape(table, (_SROWS, 128))
    p = jnp.arange(_PER_W, dtype=jnp.int32)
    pattern = ((p % _L) * _CH + p // _L).reshape(_L, _CH)

    mesh = plsc.VectorSubcoreMesh(
        core_axis_name="c", subcore_axis_name="s",
        num_cores=_NC, num_subcores=_NS)
    params = pltpu.CompilerParams(use_tc_tiling_on_sc=True,
                                  needs_layout_passes=False)

    out4 = pl.kernel(
        _k2_body,
        out_type=jax.ShapeDtypeStruct((_TOT // 4, 128), jnp.float32),
        mesh=mesh,
        scratch_types=[
            pltpu.VMEM((_L, _CH), jnp.int32),
            pltpu.VMEM((_L, _CH), jnp.int32),
            pltpu.VMEM((_NCH, _CH), jnp.int32),
            pltpu.VMEM((_NCH, _CH), jnp.int32),
            pltpu.VMEM((_CH, 128), jnp.float32),
            pltpu.VMEM((_CH, 128), jnp.float32),
            pltpu.VMEM((_CH // 4, 128), jnp.float32),
            pltpu.VMEM((_CH // 4, 128), jnp.float32),
            pltpu.SemaphoreType.DMA,
            pltpu.SemaphoreType.DMA,
            pltpu.SemaphoreType.DMA,
            pltpu.SemaphoreType.DMA,
        ],
        compiler_params=params,
    )(cond_t, scratch, pattern)

    return out4.reshape(_B, _L * _H)


# speedup vs baseline: 2.2865x; 1.3886x over previous
import jax
import jax.numpy as jnp
from jax import lax
from jax.experimental import pallas as pl
from jax.experimental.pallas import tpu as pltpu
from jax.experimental.pallas import tpu_sc as plsc

_NC = 2
_NS = 16
_NW = _NC * _NS

_B = 4096
_L = 50
_H = 32
_TOT = _B * _L
_PER_W = _TOT // _NW
_CH = 128
_NCH = _PER_W // _CH
_SUP = 5
_NSUP = _NCH // _SUP
_SROWS = _SUP * _CH


def _emb_body(cond_hbm, table_hbm, out_hbm, idx_v, buf0, buf1, g0, g1, w0, w1):
    wid = lax.axis_index("s") * _NC + lax.axis_index("c")
    out_w = out_hbm.at[wid]
    pltpu.sync_copy(cond_hbm.at[wid], idx_v)

    def fire(buf, gsem, s):
        for k in range(_SUP):
            pltpu.async_copy(
                table_hbm.at[idx_v.at[s * _SUP + k]],
                buf.at[pl.ds(k * _CH, _CH)], gsem)

    def drain_gather(buf, gsem, s):
        pltpu.make_async_copy(out_w.at[s], buf, gsem).wait()

    def start_write(buf, wsem, s):
        pltpu.async_copy(buf, out_w.at[s], wsem)

    def wait_write(buf, wsem, s):
        pltpu.make_async_copy(buf, out_w.at[s], wsem).wait()

    def outer(t, carry):
        s0 = 2 * t
        s1 = 2 * t + 1

        @pl.when(t > 0)
        def _():
            wait_write(buf0, w0, s0 - 2)
        fire(buf0, g0, s0)

        @pl.when(t > 0)
        def _():
            drain_gather(buf1, g1, s0 - 1)
            start_write(buf1, w1, s0 - 1)
            wait_write(buf1, w1, s1 - 2)
        fire(buf1, g1, s1)

        drain_gather(buf0, g0, s0)
        start_write(buf0, w0, s0)
        return carry

    lax.fori_loop(0, _NSUP // 2, outer, 0)
    drain_gather(buf1, g1, _NSUP - 1)
    start_write(buf1, w1, _NSUP - 1)
    wait_write(buf0, w0, _NSUP - 2)
    wait_write(buf1, w1, _NSUP - 1)


@jax.jit
def kernel(conditions, table):
    idx = conditions.reshape(_NW, _NCH, _CH)
    mesh = plsc.VectorSubcoreMesh(
        core_axis_name="c", subcore_axis_name="s",
        num_cores=_NC, num_subcores=_NS)
    out = pl.kernel(
        _emb_body,
        out_type=jax.ShapeDtypeStruct((_NW, _NSUP, _SROWS, _H), jnp.float32),
        mesh=mesh,
        scratch_types=[
            pltpu.VMEM((_NCH, _CH), jnp.int32),
            pltpu.VMEM((_SROWS, _H), jnp.float32),
            pltpu.VMEM((_SROWS, _H), jnp.float32),
            pltpu.SemaphoreType.DMA,
            pltpu.SemaphoreType.DMA,
            pltpu.SemaphoreType.DMA,
            pltpu.SemaphoreType.DMA,
        ],
        compiler_params=pltpu.CompilerParams(use_tc_tiling_on_sc=False),
    )(idx, table)
    return out.reshape(_B, _L * _H)
